# baseline (speedup 1.0000x reference)
import functools

import jax
import jax.numpy as jnp
from jax import lax
from jax.experimental import pallas as pl
from jax.experimental.pallas import tpu as pltpu
from jax.experimental.pallas import tpu_sc as plsc

B = 4096
C = 1000
D = 512
LAMBDA1 = 0.005
LAMBDA2 = 1.0

PTC = 2560
BBT = 512
CP = 1024

NC = 2
NS = 16
NW = NC * NS
QSC = B - PTC
BPW = QSC // NW
R = 24
NCH = BPW // R
LN = 16
DV = D // LN

_mesh = plsc.VectorSubcoreMesh(core_axis_name="c", subcore_axis_name="s")


@functools.partial(
    pl.kernel,
    out_type=jax.ShapeDtypeStruct((NW * LN,), jnp.float32),
    mesh=_mesh,
    scratch_types=[
        pltpu.VMEM((BPW + LN,), jnp.int32),
        pltpu.VMEM((2, R, D), jnp.float32),
        pltpu.VMEM((2, R, D), jnp.float32),
        pltpu.VMEM((CP,), jnp.float32),
        pltpu.VMEM((LN,), jnp.float32),
        pltpu.SemaphoreType.DMA((2,)),
        pltpu.SemaphoreType.DMA((2,)),
    ],
)
def _sc_loss(x_hbm, y_hbm, w_hbm, g2_hbm, out_hbm,
             idx_v, xbuf, wbuf, gpv, stage, semx, semw):
    wid = lax.axis_index("s") * NC + lax.axis_index("c")
    base = PTC + wid * BPW

    pltpu.sync_copy(y_hbm.at[pl.ds(base, BPW)], idx_v.at[pl.ds(0, BPW)])
    pltpu.sync_copy(g2_hbm, gpv.at[pl.ds(0, C)])

    def start_chunk(c, b):
        return (
            pltpu.async_copy(x_hbm.at[pl.ds(base + c * R, R)], xbuf.at[b],
                             semx.at[b]),
            pltpu.async_copy(w_hbm.at[idx_v.at[pl.ds(c * R, R)]], wbuf.at[b],
                             semw.at[b]),
        )

    inflight = [None, None]
    inflight[0] = start_chunk(0, 0)
    zero = jnp.zeros((LN,), jnp.float32)
    accs = (zero, zero, zero, zero)
    for c in range(NCH):
        b = c & 1
        if c + 1 < NCH:
            inflight[(c + 1) & 1] = start_chunk(c + 1, (c + 1) & 1)
        for cp in inflight[b]:
            cp.wait()
        xb, wb = xbuf.at[b], wbuf.at[b]

        def row_body(r, carry):
            yr = idx_v[pl.ds(c * R + r, LN)][0]
            g16 = jnp.full((LN,), gpv[pl.ds(yr, LN)][0], jnp.float32)
            a0, a1, a2, a3 = carry
            for dd in range(0, DV, 4):
                t0 = xb[r, pl.ds(dd * LN, LN)] - g16 * wb[r, pl.ds(dd * LN, LN)]
                t1 = xb[r, pl.ds((dd + 1) * LN, LN)] - g16 * wb[r, pl.ds((dd + 1) * LN, LN)]
                t2 = xb[r, pl.ds((dd + 2) * LN, LN)] - g16 * wb[r, pl.ds((dd + 2) * LN, LN)]
                t3 = xb[r, pl.ds((dd + 3) * LN, LN)] - g16 * wb[r, pl.ds((dd + 3) * LN, LN)]
                a0 = a0 + t0 * t0
                a1 = a1 + t1 * t1
                a2 = a2 + t2 * t2
                a3 = a3 + t3 * t3
            return a0, a1, a2, a3

        accs = lax.fori_loop(0, R, row_body, accs)

    stage[...] = (accs[0] + accs[1]) + (accs[2] + accs[3])
    pltpu.sync_copy(stage, out_hbm.at[pl.ds(wid * LN, LN)])


def _tc_body(y_ref, x_ref, w_ref, g_ref, out_ref, s_acc, ssq):
    pid = pl.program_id(0)
    nb = pl.num_programs(0)

    @pl.when(pid == 0)
    def _():
        s_acc[...] = jnp.zeros_like(s_acc)
        ssq[0] = 0.0

    xb = x_ref[...]
    yb = y_ref[...]
    ssq[0] += jnp.sum(xb * xb)
    iot = lax.broadcasted_iota(jnp.int32, (CP, BBT), 0)
    oh = (iot == yb[None, :]).astype(jnp.bfloat16)
    xaug = jnp.concatenate(
        [xb.astype(jnp.bfloat16), jnp.ones((BBT, 128), jnp.bfloat16)], axis=1)
    s_acc[...] += lax.dot_general(oh, xaug,
                                  (((1,), (0,)), ((), ())),
                                  preferred_element_type=jnp.float32)

    @pl.when(pid == nb - 1)
    def _():
        w = w_ref[...]
        g = g_ref[...]
        sl = s_acc[pl.ds(0, C), pl.ds(0, D)]
        nl = s_acc[pl.ds(0, C), pl.ds(D, 128)][:, 0:1]
        term2 = jnp.sum(sl * w * g)
        w2 = jnp.sum(w * w, axis=1, keepdims=True)
        term3 = jnp.sum(nl * g * g * w2)
        sg = jnp.sum(g)
        sg2 = jnp.sum(g * g)
        vals = [ssq[0] - 2.0 * term2 + term3, sg, sg2]
        out_ref[...] = jnp.concatenate(
            [jnp.full((1, 128), v, jnp.float32) for v in vals], axis=0)


_tc_part = pl.pallas_call(
    _tc_body,
    grid=(PTC // BBT,),
    in_specs=[
        pl.BlockSpec((BBT,), lambda i: (i,)),
        pl.BlockSpec((BBT, D), lambda i: (i, 0)),
        pl.BlockSpec((C, D), lambda i: (0, 0)),
        pl.BlockSpec((C, 1), lambda i: (0, 0)),
    ],
    out_specs=pl.BlockSpec((3, 128), lambda i: (0, 0)),
    out_shape=jax.ShapeDtypeStruct((3, 128), jnp.float32),
    scratch_shapes=[
        pltpu.VMEM((CP, D + 128), jnp.float32),
        pltpu.SMEM((1,), jnp.float32),
    ],
)


def kernel(output_features, y_truth, fixed_weights, centers_gamma):
    gflat = centers_gamma.reshape(-1)
    out_sc = _sc_loss(output_features, y_truth, fixed_weights, gflat)
    out_tc = _tc_part(y_truth, output_features, fixed_weights, centers_gamma)

    loss1 = 0.5 * (jnp.sum(out_sc) + out_tc[0, 0]) / B
    sg = out_tc[1, 0]
    sg2 = out_tc[2, 0]
    L = 2.0 * (C - 1) * sg2 + 2.0 * (sg * sg - sg2) / (C - 1)
    loss2 = C * (C - 1) / L
    return LAMBDA1 * loss1 + LAMBDA2 * loss2


# speedup vs baseline: 1.0846x; 1.0846x over previous
import functools

import jax
import jax.numpy as jnp
from jax import lax
from jax.experimental import pallas as pl
from jax.experimental.pallas import tpu as pltpu
from jax.experimental.pallas import tpu_sc as plsc

B = 4096
C = 1000
D = 512
LAMBDA1 = 0.005
LAMBDA2 = 1.0

PTC = 2048
PTA = 2048
CP = 1024

NC = 2
NS = 16
NW = NC * NS
QSC = B - PTC
BPW = QSC // NW
R = 32
NCH = BPW // R
LN = 16
DV = D // LN

_mesh = plsc.VectorSubcoreMesh(core_axis_name="c", subcore_axis_name="s")


@functools.partial(
    pl.kernel,
    out_type=jax.ShapeDtypeStruct((NW * LN,), jnp.float32),
    mesh=_mesh,
    scratch_types=[
        pltpu.VMEM((BPW + LN,), jnp.int32),
        pltpu.VMEM((2, R, D), jnp.float32),
        pltpu.VMEM((2, R, D), jnp.float32),
        pltpu.VMEM((CP,), jnp.float32),
        pltpu.VMEM((LN,), jnp.float32),
        pltpu.SemaphoreType.DMA((2,)),
        pltpu.SemaphoreType.DMA((2,)),
    ],
)
def _sc_loss(x_hbm, y_hbm, w_hbm, g2_hbm, out_hbm,
             idx_v, xbuf, wbuf, gpv, stage, semx, semw):
    wid = lax.axis_index("s") * NC + lax.axis_index("c")
    base = PTC + wid * BPW

    xcopies = [
        pltpu.async_copy(x_hbm.at[pl.ds(base + c * R, R)], xbuf.at[c & 1],
                         semx.at[c & 1])
        for c in range(min(NCH, 2))
    ]
    pltpu.sync_copy(y_hbm.at[pl.ds(base, BPW)], idx_v.at[pl.ds(0, BPW)])

    def start_chunk(c, b, first=False):
        return (
            xcopies[c] if first else
            pltpu.async_copy(x_hbm.at[pl.ds(base + c * R, R)], xbuf.at[b],
                             semx.at[b]),
            pltpu.async_copy(w_hbm.at[idx_v.at[pl.ds(c * R, R)]], wbuf.at[b],
                             semw.at[b]),
        )

    inflight = [None, None]
    inflight[0] = start_chunk(0, 0, first=True)
    if NCH > 1:
        inflight[1] = start_chunk(1, 1, first=True)
    pltpu.sync_copy(g2_hbm, gpv.at[pl.ds(0, C)])
    zero = jnp.zeros((LN,), jnp.float32)
    accs = (zero, zero, zero, zero)
    for c in range(NCH):
        b = c & 1
        if c + 1 < NCH and c > 0:
            inflight[(c + 1) & 1] = start_chunk(c + 1, (c + 1) & 1)
        for cp in inflight[b]:
            cp.wait()
        xb, wb = xbuf.at[b], wbuf.at[b]

        @plsc.parallel_loop(0, R, step=1, carry=accs)
        def accs(r, carry):
            yr = idx_v[pl.ds(c * R + r, LN)][0]
            g16 = jnp.full((LN,), gpv[pl.ds(yr, LN)][0], jnp.float32)
            a0, a1, a2, a3 = carry
            for dd in range(0, DV, 4):
                t0 = xb[r, pl.ds(dd * LN, LN)] - g16 * wb[r, pl.ds(dd * LN, LN)]
                t1 = xb[r, pl.ds((dd + 1) * LN, LN)] - g16 * wb[r, pl.ds((dd + 1) * LN, LN)]
                t2 = xb[r, pl.ds((dd + 2) * LN, LN)] - g16 * wb[r, pl.ds((dd + 2) * LN, LN)]
                t3 = xb[r, pl.ds((dd + 3) * LN, LN)] - g16 * wb[r, pl.ds((dd + 3) * LN, LN)]
                a0 = a0 + t0 * t0
                a1 = a1 + t1 * t1
                a2 = a2 + t2 * t2
                a3 = a3 + t3 * t3
            return a0, a1, a2, a3

    stage[...] = (accs[0] + accs[1]) + (accs[2] + accs[3])
    pltpu.sync_copy(stage, out_hbm.at[pl.ds(wid * LN, LN)])


def _make_tc_part(row_off, rows, bbt):
    blk_off = row_off // bbt

    def body(y_ref, x_ref, w_ref, g_ref, out_ref, s_acc, ssq):
        pid = pl.program_id(0)
        nb = pl.num_programs(0)

        @pl.when(pid == 0)
        def _():
            s_acc[...] = jnp.zeros_like(s_acc)
            ssq[0] = 0.0

        xb = x_ref[...]
        yb = y_ref[...]
        ssq[0] += jnp.sum(xb * xb)
        iot = lax.broadcasted_iota(jnp.int32, (CP, bbt), 0)
        oh = (iot == yb[None, :]).astype(jnp.bfloat16)
        xaug = jnp.concatenate(
            [xb.astype(jnp.bfloat16), jnp.ones((bbt, 128), jnp.bfloat16)],
            axis=1)
        s_acc[...] += lax.dot_general(oh, xaug,
                                      (((1,), (0,)), ((), ())),
                                      preferred_element_type=jnp.float32)

        @pl.when(pid == nb - 1)
        def _():
            w = w_ref[...]
            g = g_ref[...]
            sl = s_acc[pl.ds(0, C), pl.ds(0, D)]
            nl = s_acc[pl.ds(0, C), pl.ds(D, 128)][:, 0:1]
            term2 = jnp.sum(sl * w * g)
            w2 = jnp.sum(w * w, axis=1, keepdims=True)
            term3 = jnp.sum(nl * g * g * w2)
            sg = jnp.sum(g)
            sg2 = jnp.sum(g * g)
            vals = [ssq[0] - 2.0 * term2 + term3, sg, sg2]
            out_ref[...] = jnp.concatenate(
                [jnp.full((1, 128), v, jnp.float32) for v in vals], axis=0)

    return pl.pallas_call(
        body,
        grid=(rows // bbt,),
        in_specs=[
            pl.BlockSpec((bbt,), lambda i: (i + blk_off,)),
            pl.BlockSpec((bbt, D), lambda i: (i + blk_off, 0)),
            pl.BlockSpec((C, D), lambda i: (0, 0)),
            pl.BlockSpec((C, 1), lambda i: (0, 0)),
        ],
        out_specs=pl.BlockSpec((3, 128), lambda i: (0, 0)),
        out_shape=jax.ShapeDtypeStruct((3, 128), jnp.float32),
        scratch_shapes=[
            pltpu.VMEM((CP, D + 128), jnp.float32),
            pltpu.SMEM((1,), jnp.float32),
        ],
    )


_tc_part = _make_tc_part(0, PTC, PTA)


def kernel(output_features, y_truth, fixed_weights, centers_gamma):
    gflat = centers_gamma.reshape(-1)
    out_sc = _sc_loss(output_features, y_truth, fixed_weights, gflat)
    out_tc = _tc_part(y_truth, output_features, fixed_weights, centers_gamma)

    loss1 = 0.5 * (jnp.sum(out_sc) + out_tc[0, 0]) / B
    sg = out_tc[1, 0]
    sg2 = out_tc[2, 0]
    L = 2.0 * (C - 1) * sg2 + 2.0 * (sg * sg - sg2) / (C - 1)
    loss2 = C * (C - 1) / L
    return LAMBDA1 * loss1 + LAMBDA2 * loss2
